# baseline (speedup 1.0000x reference)
import functools

import numpy as np
import jax
import jax.numpy as jnp
from jax import lax
from jax.experimental import pallas as pl
from jax.experimental.pallas import tpu as pltpu
from jax.experimental.pallas import tpu_sc as plsc

_N = 10000
_D = 128
_NCORES = 2
_NSUB = 16
_NTILES = _NCORES * _NSUB
_C = 80
_NBUF = 3
_STRIPE = 624
_TAIL = _N - _NSUB * _STRIPE


def _edge_mul_body(m_ref, rbf_ref, w_ref, x_ref):
    x_ref[...] = m_ref[...] * jnp.dot(
        rbf_ref[...], w_ref[...], preferred_element_type=jnp.float32)


def _edge_mul(m, rbf, w_rbf, start, size):
    be = 2560
    sb = start // be
    return pl.pallas_call(
        _edge_mul_body,
        grid=(size // be,),
        in_specs=[
            pl.BlockSpec((be, _D), lambda i: (sb + i, 0)),
            pl.BlockSpec((be, w_rbf.shape[0]), lambda i: (sb + i, 0)),
            pl.BlockSpec(w_rbf.shape, lambda i: (0, 0)),
        ],
        out_specs=pl.BlockSpec((be, _D), lambda i: (i, 0)),
        out_shape=jax.ShapeDtypeStruct((size, _D), jnp.float32),
    )(m, rbf, w_rbf)


def _seg_sum_body(nch, et, split_base, x_hbm, idx_hbm, z_hbm, out_hbm,
                  acc, idxb, xb, semi, semx, sems):
    c = lax.axis_index("c")
    s = lax.axis_index("s")
    tile = c * _NSUB + s
    base = tile * et
    ibase = split_base + base
    rows0 = s * _STRIPE

    pltpu.sync_copy(z_hbm.at[pl.ds(0, _STRIPE)], acc.at[pl.ds(rows0, _STRIPE)])

    @pl.when(s == _NSUB - 1)
    def _():
        pltpu.sync_copy(z_hbm.at[pl.ds(0, _TAIL)],
                        acc.at[pl.ds(_NSUB * _STRIPE, _TAIL)])

    plsc.subcore_barrier()

    def _scatter_desc(b):
        return pltpu.make_async_copy(xb.at[b], acc.at[idxb.at[b]],
                                     sems.at[b])

    def _start(ck, b, drain):
        if drain:
            _scatter_desc(b).wait()
        pltpu.async_copy(idx_hbm.at[pl.ds(ibase + ck * _C, _C)], idxb.at[b],
                         semi.at[b])
        pltpu.async_copy(x_hbm.at[pl.ds(base + ck * _C, _C)], xb.at[b],
                         semx.at[b])

    def _consume(ck, b):
        pltpu.make_async_copy(idx_hbm.at[pl.ds(ibase + ck * _C, _C)],
                              idxb.at[b], semi.at[b]).wait()
        pltpu.make_async_copy(x_hbm.at[pl.ds(base + ck * _C, _C)],
                              xb.at[b], semx.at[b]).wait()
        pltpu.async_copy(xb.at[b], acc.at[idxb.at[b]], sems.at[b], add=True)

    for b in range(min(_NBUF, nch)):
        _start(b, b, drain=False)

    @pl.loop(0, (nch // _NBUF) * _NBUF, step=_NBUF)
    def _chunks(k):
        for b in range(_NBUF):
            ck = k + b
            _consume(ck, b)
            nk = ck + _NBUF

            @pl.when(nk < nch)
            def _():
                _start(nk, b, drain=True)

    for r in range((nch // _NBUF) * _NBUF, nch):
        _consume(r, r % _NBUF)

    for b in range(min(_NBUF, nch)):
        _scatter_desc(b).wait()

    plsc.subcore_barrier()
    pltpu.sync_copy(acc.at[pl.ds(rows0, _STRIPE)],
                    out_hbm.at[c, pl.ds(rows0, _STRIPE)])

    @pl.when(s == _NSUB - 1)
    def _():
        pltpu.sync_copy(acc.at[pl.ds(_NSUB * _STRIPE, _TAIL)],
                        out_hbm.at[c, pl.ds(_NSUB * _STRIPE, _TAIL)])


def _seg_sum(x, ids, split_base):
    e = x.shape[0]
    et = e // _NTILES
    nch = et // _C
    zeros = jnp.zeros((_STRIPE, _D), jnp.float32)
    body = functools.partial(_seg_sum_body, nch, et, split_base)
    return pl.kernel(
        body,
        out_type=jax.ShapeDtypeStruct((_NCORES, _N, _D), jnp.float32),
        mesh=plsc.VectorSubcoreMesh(core_axis_name="c", subcore_axis_name="s"),
        scratch_types=[
            pltpu.VMEM_SHARED((_N, _D), jnp.float32),
            pltpu.VMEM((_NBUF, _C), jnp.int32),
            pltpu.VMEM((_NBUF, _C, _D), jnp.float32),
            pltpu.SemaphoreType.DMA((_NBUF,)),
            pltpu.SemaphoreType.DMA((_NBUF,)),
            pltpu.SemaphoreType.DMA((_NBUF,)),
        ],
    )(x, ids, zeros)


def _mlp_body(n_hidden, n_parts, scale_ref, *refs):
    p_refs, (w1_ref, rw_ref, o_ref) = refs[:n_parts], refs[n_parts:]
    inv_sqrt2 = np.float32(1.0 / np.sqrt(2.0))
    tot = p_refs[0][0] + p_refs[0][1]
    for p in p_refs[1:]:
        tot = tot + p[0] + p[1]
    x2 = tot * scale_ref[0]
    h = jax.nn.silu(jnp.dot(x2, w1_ref[...],
                            preferred_element_type=jnp.float32))
    for i in range(n_hidden):
        r = jax.nn.silu(jnp.dot(h, rw_ref[i, 0],
                                preferred_element_type=jnp.float32))
        r = jax.nn.silu(jnp.dot(r, rw_ref[i, 1],
                                preferred_element_type=jnp.float32))
        h = (h + r) * inv_sqrt2
    o_ref[...] = h


def _mlp(parts, w1, res_w, scale):
    bn = 1000
    n_hidden = res_w.shape[0]
    return pl.pallas_call(
        functools.partial(_mlp_body, n_hidden, len(parts)),
        grid=(_N // bn,),
        in_specs=[
            pl.BlockSpec(memory_space=pltpu.SMEM),
            *[pl.BlockSpec((_NCORES, bn, _D), lambda i: (0, i, 0))
              for _ in parts],
            pl.BlockSpec((_D, _D), lambda i: (0, 0)),
            pl.BlockSpec(res_w.shape, lambda i: (0, 0, 0, 0)),
        ],
        out_specs=pl.BlockSpec((bn, _D), lambda i: (i, 0)),
        out_shape=jax.ShapeDtypeStruct((_N, _D), jnp.float32),
    )(scale.reshape(1), *parts, w1, res_w)


_SPLITS = (81920, 79360, 79360, 79360)


def kernel(nAtoms, m, rbf, id_j, W_rbf, W1, res_w, scale):
    ids = jnp.remainder(id_j.astype(jnp.int32), nAtoms).astype(jnp.int32)
    parts = []
    start = 0
    for size in _SPLITS:
        x = _edge_mul(m, rbf, W_rbf, start, size)
        parts.append(_seg_sum(x, ids, start))
        start += size
    return _mlp(parts, W1, res_w, scale)


# speedup vs baseline: 1.0405x; 1.0405x over previous
import functools

import numpy as np
import jax
import jax.numpy as jnp
from jax import lax
from jax.experimental import pallas as pl
from jax.experimental.pallas import tpu as pltpu
from jax.experimental.pallas import tpu_sc as plsc

_N = 10000
_D = 128
_NCORES = 2
_NSUB = 16
_NTILES = _NCORES * _NSUB
_C = 80
_NBUF = 3
_STRIPE = 624
_TAIL = _N - _NSUB * _STRIPE


def _edge_mul_body(m_ref, rbf_ref, w_ref, x_ref):
    x_ref[...] = m_ref[...] * jnp.dot(
        rbf_ref[...], w_ref[...], preferred_element_type=jnp.float32)


def _edge_mul(m, rbf, w_rbf, start, size):
    be = 2560
    sb = start // be
    return pl.pallas_call(
        _edge_mul_body,
        grid=(size // be,),
        in_specs=[
            pl.BlockSpec((be, _D), lambda i: (sb + i, 0)),
            pl.BlockSpec((be, w_rbf.shape[0]), lambda i: (sb + i, 0)),
            pl.BlockSpec(w_rbf.shape, lambda i: (0, 0)),
        ],
        out_specs=pl.BlockSpec((be, _D), lambda i: (i, 0)),
        out_shape=jax.ShapeDtypeStruct((size, _D), jnp.float32),
    )(m, rbf, w_rbf)


def _seg_sum_body(nch, et, split_base, x_hbm, idx_hbm, z_hbm, out_hbm,
                  acc, idxb, xb, semi, semx, sems):
    c = lax.axis_index("c")
    s = lax.axis_index("s")
    tile = c * _NSUB + s
    base = tile * et
    ibase = split_base + base
    rows0 = s * _STRIPE

    pltpu.sync_copy(z_hbm.at[pl.ds(0, _STRIPE)], acc.at[pl.ds(rows0, _STRIPE)])

    @pl.when(s == _NSUB - 1)
    def _():
        pltpu.sync_copy(z_hbm.at[pl.ds(0, _TAIL)],
                        acc.at[pl.ds(_NSUB * _STRIPE, _TAIL)])

    plsc.subcore_barrier()

    def _scatter_desc(b):
        return pltpu.make_async_copy(xb.at[b], acc.at[idxb.at[b]],
                                     sems.at[b])

    def _start(ck, b, drain):
        if drain:
            _scatter_desc(b).wait()
        pltpu.async_copy(idx_hbm.at[pl.ds(ibase + ck * _C, _C)], idxb.at[b],
                         semi.at[b])
        pltpu.async_copy(x_hbm.at[pl.ds(base + ck * _C, _C)], xb.at[b],
                         semx.at[b])

    def _consume(ck, b):
        pltpu.make_async_copy(idx_hbm.at[pl.ds(ibase + ck * _C, _C)],
                              idxb.at[b], semi.at[b]).wait()
        pltpu.make_async_copy(x_hbm.at[pl.ds(base + ck * _C, _C)],
                              xb.at[b], semx.at[b]).wait()
        pltpu.async_copy(xb.at[b], acc.at[idxb.at[b]], sems.at[b], add=True)

    for b in range(min(_NBUF, nch)):
        _start(b, b, drain=False)

    @pl.loop(0, (nch // _NBUF) * _NBUF, step=_NBUF)
    def _chunks(k):
        for b in range(_NBUF):
            ck = k + b
            _consume(ck, b)
            nk = ck + _NBUF

            @pl.when(nk < nch)
            def _():
                _start(nk, b, drain=True)

    for r in range((nch // _NBUF) * _NBUF, nch):
        _consume(r, r % _NBUF)

    for b in range(min(_NBUF, nch)):
        _scatter_desc(b).wait()

    plsc.subcore_barrier()
    pltpu.sync_copy(acc.at[pl.ds(rows0, _STRIPE)],
                    out_hbm.at[c, pl.ds(rows0, _STRIPE)])

    @pl.when(s == _NSUB - 1)
    def _():
        pltpu.sync_copy(acc.at[pl.ds(_NSUB * _STRIPE, _TAIL)],
                        out_hbm.at[c, pl.ds(_NSUB * _STRIPE, _TAIL)])


def _seg_sum(x, ids, split_base):
    e = x.shape[0]
    et = e // _NTILES
    nch = et // _C
    zeros = jnp.zeros((_STRIPE, _D), jnp.float32)
    body = functools.partial(_seg_sum_body, nch, et, split_base)
    return pl.kernel(
        body,
        out_type=jax.ShapeDtypeStruct((_NCORES, _N, _D), jnp.float32),
        mesh=plsc.VectorSubcoreMesh(core_axis_name="c", subcore_axis_name="s"),
        scratch_types=[
            pltpu.VMEM_SHARED((_N, _D), jnp.float32),
            pltpu.VMEM((_NBUF, _C), jnp.int32),
            pltpu.VMEM((_NBUF, _C, _D), jnp.float32),
            pltpu.SemaphoreType.DMA((_NBUF,)),
            pltpu.SemaphoreType.DMA((_NBUF,)),
            pltpu.SemaphoreType.DMA((_NBUF,)),
        ],
    )(x, ids, zeros)


def _mlp_body(n_hidden, n_parts, scale_ref, *refs):
    p_refs, (w1_ref, rw_ref, o_ref) = refs[:n_parts], refs[n_parts:]
    inv_sqrt2 = np.float32(1.0 / np.sqrt(2.0))
    tot = p_refs[0][0] + p_refs[0][1]
    for p in p_refs[1:]:
        tot = tot + p[0] + p[1]
    x2 = tot * scale_ref[0]
    h = jax.nn.silu(jnp.dot(x2, w1_ref[...],
                            preferred_element_type=jnp.float32))
    for i in range(n_hidden):
        r = jax.nn.silu(jnp.dot(h, rw_ref[i, 0],
                                preferred_element_type=jnp.float32))
        r = jax.nn.silu(jnp.dot(r, rw_ref[i, 1],
                                preferred_element_type=jnp.float32))
        h = (h + r) * inv_sqrt2
    o_ref[...] = h


def _mlp(parts, w1, res_w, scale):
    bn = 1000
    n_hidden = res_w.shape[0]
    return pl.pallas_call(
        functools.partial(_mlp_body, n_hidden, len(parts)),
        grid=(_N // bn,),
        in_specs=[
            pl.BlockSpec(memory_space=pltpu.SMEM),
            *[pl.BlockSpec((_NCORES, bn, _D), lambda i: (0, i, 0))
              for _ in parts],
            pl.BlockSpec((_D, _D), lambda i: (0, 0)),
            pl.BlockSpec(res_w.shape, lambda i: (0, 0, 0, 0)),
        ],
        out_specs=pl.BlockSpec((bn, _D), lambda i: (i, 0)),
        out_shape=jax.ShapeDtypeStruct((_N, _D), jnp.float32),
    )(scale.reshape(1), *parts, w1, res_w)


_SPLITS = (161280, 158720)


def kernel(nAtoms, m, rbf, id_j, W_rbf, W1, res_w, scale):
    ids = jnp.remainder(id_j.astype(jnp.int32), nAtoms).astype(jnp.int32)
    parts = []
    start = 0
    for size in _SPLITS:
        x = _edge_mul(m, rbf, W_rbf, start, size)
        parts.append(_seg_sum(x, ids, start))
        start += size
    return _mlp(parts, W1, res_w, scale)


# speedup vs baseline: 1.0652x; 1.0237x over previous
import functools

import numpy as np
import jax
import jax.numpy as jnp
from jax import lax
from jax.experimental import pallas as pl
from jax.experimental.pallas import tpu as pltpu
from jax.experimental.pallas import tpu_sc as plsc

_N = 10000
_D = 128
_NCORES = 2
_NSUB = 16
_NTILES = _NCORES * _NSUB
_C = 80
_NBUF = 3
_STRIPE = 624
_TAIL = _N - _NSUB * _STRIPE


def _edge_mul_body(m_ref, rbf_ref, w_ref, x_ref):
    x_ref[...] = m_ref[...] * jnp.dot(
        rbf_ref[...], w_ref[...], preferred_element_type=jnp.float32)


def _edge_mul(m, rbf, w_rbf, start, size):
    be = 2560
    sb = start // be
    return pl.pallas_call(
        _edge_mul_body,
        grid=(size // be,),
        in_specs=[
            pl.BlockSpec((be, _D), lambda i: (sb + i, 0)),
            pl.BlockSpec((be, w_rbf.shape[0]), lambda i: (sb + i, 0)),
            pl.BlockSpec(w_rbf.shape, lambda i: (0, 0)),
        ],
        out_specs=pl.BlockSpec((be, _D), lambda i: (i, 0)),
        out_shape=jax.ShapeDtypeStruct((size, _D), jnp.float32),
    )(m, rbf, w_rbf)


def _seg_sum_body(nch, et, split_base, x_hbm, idx_hbm, out_hbm,
                  acc, idxb, xb, semi, semx, sems):
    c = lax.axis_index("c")
    s = lax.axis_index("s")
    tile = c * _NSUB + s
    base = tile * et
    ibase = split_base + base
    rows0 = s * _STRIPE

    @pl.loop(0, _C)
    def _zrow(r):
        for g in range(_D // 16):
            xb[0, r, pl.ds(g * 16, 16)] = jnp.zeros((16,), jnp.float32)

    for k in range(_STRIPE // _C):
        pltpu.sync_copy(xb.at[0], acc.at[pl.ds(rows0 + k * _C, _C)])
    rem = _STRIPE % _C
    if rem:
        pltpu.sync_copy(xb.at[0, pl.ds(0, rem)],
                        acc.at[pl.ds(rows0 + (_STRIPE // _C) * _C, rem)])

    @pl.when(s == _NSUB - 1)
    def _():
        pltpu.sync_copy(xb.at[0, pl.ds(0, _TAIL)],
                        acc.at[pl.ds(_NSUB * _STRIPE, _TAIL)])

    plsc.subcore_barrier()

    def _scatter_desc(b):
        return pltpu.make_async_copy(xb.at[b], acc.at[idxb.at[b]],
                                     sems.at[b])

    def _start(ck, b, drain):
        if drain:
            _scatter_desc(b).wait()
        pltpu.async_copy(idx_hbm.at[pl.ds(ibase + ck * _C, _C)], idxb.at[b],
                         semi.at[b])
        pltpu.async_copy(x_hbm.at[pl.ds(base + ck * _C, _C)], xb.at[b],
                         semx.at[b])

    def _consume(ck, b):
        pltpu.make_async_copy(idx_hbm.at[pl.ds(ibase + ck * _C, _C)],
                              idxb.at[b], semi.at[b]).wait()
        pltpu.make_async_copy(x_hbm.at[pl.ds(base + ck * _C, _C)],
                              xb.at[b], semx.at[b]).wait()
        pltpu.async_copy(xb.at[b], acc.at[idxb.at[b]], sems.at[b], add=True)

    for b in range(min(_NBUF, nch)):
        _start(b, b, drain=False)

    @pl.loop(0, (nch // _NBUF) * _NBUF, step=_NBUF)
    def _chunks(k):
        for b in range(_NBUF):
            ck = k + b
            _consume(ck, b)
            nk = ck + _NBUF

            @pl.when(nk < nch)
            def _():
                _start(nk, b, drain=True)

    for r in range((nch // _NBUF) * _NBUF, nch):
        _consume(r, r % _NBUF)

    for b in range(min(_NBUF, nch)):
        _scatter_desc(b).wait()

    plsc.subcore_barrier()
    pltpu.sync_copy(acc.at[pl.ds(rows0, _STRIPE)],
                    out_hbm.at[c, pl.ds(rows0, _STRIPE)])

    @pl.when(s == _NSUB - 1)
    def _():
        pltpu.sync_copy(acc.at[pl.ds(_NSUB * _STRIPE, _TAIL)],
                        out_hbm.at[c, pl.ds(_NSUB * _STRIPE, _TAIL)])


def _seg_sum(x, ids, split_base):
    e = x.shape[0]
    et = e // _NTILES
    nch = et // _C
    body = functools.partial(_seg_sum_body, nch, et, split_base)
    return pl.kernel(
        body,
        out_type=jax.ShapeDtypeStruct((_NCORES, _N, _D), jnp.float32),
        mesh=plsc.VectorSubcoreMesh(core_axis_name="c", subcore_axis_name="s"),
        scratch_types=[
            pltpu.VMEM_SHARED((_N, _D), jnp.float32),
            pltpu.VMEM((_NBUF, _C), jnp.int32),
            pltpu.VMEM((_NBUF, _C, _D), jnp.float32),
            pltpu.SemaphoreType.DMA((_NBUF,)),
            pltpu.SemaphoreType.DMA((_NBUF,)),
            pltpu.SemaphoreType.DMA((_NBUF,)),
        ],
    )(x, ids)


def _mlp_body(n_hidden, n_parts, scale_ref, *refs):
    p_refs, (w1_ref, rw_ref, o_ref) = refs[:n_parts], refs[n_parts:]
    inv_sqrt2 = np.float32(1.0 / np.sqrt(2.0))
    tot = p_refs[0][0] + p_refs[0][1]
    for p in p_refs[1:]:
        tot = tot + p[0] + p[1]
    x2 = tot * scale_ref[0]
    h = jax.nn.silu(jnp.dot(x2, w1_ref[...],
                            preferred_element_type=jnp.float32))
    for i in range(n_hidden):
        r = jax.nn.silu(jnp.dot(h, rw_ref[i, 0],
                                preferred_element_type=jnp.float32))
        r = jax.nn.silu(jnp.dot(r, rw_ref[i, 1],
                                preferred_element_type=jnp.float32))
        h = (h + r) * inv_sqrt2
    o_ref[...] = h


def _mlp(parts, w1, res_w, scale):
    bn = 1000
    n_hidden = res_w.shape[0]
    return pl.pallas_call(
        functools.partial(_mlp_body, n_hidden, len(parts)),
        grid=(_N // bn,),
        in_specs=[
            pl.BlockSpec(memory_space=pltpu.SMEM),
            *[pl.BlockSpec((_NCORES, bn, _D), lambda i: (0, i, 0))
              for _ in parts],
            pl.BlockSpec((_D, _D), lambda i: (0, 0)),
            pl.BlockSpec(res_w.shape, lambda i: (0, 0, 0, 0)),
        ],
        out_specs=pl.BlockSpec((bn, _D), lambda i: (i, 0)),
        out_shape=jax.ShapeDtypeStruct((_N, _D), jnp.float32),
    )(scale.reshape(1), *parts, w1, res_w)


_SPLITS = (161280, 158720)


def kernel(nAtoms, m, rbf, id_j, W_rbf, W1, res_w, scale):
    ids = jnp.remainder(id_j.astype(jnp.int32), nAtoms).astype(jnp.int32)
    parts = []
    start = 0
    for size in _SPLITS:
        x = _edge_mul(m, rbf, W_rbf, start, size)
        parts.append(_seg_sum(x, ids, start))
        start += size
    return _mlp(parts, W1, res_w, scale)
